# baseline (device time: 220114 ns/iter reference)
import jax
import jax.numpy as jnp
from jax import lax
from jax.experimental import pallas as pl
from jax.experimental.pallas import tpu as pltpu

M = 8192
N = 2048
H = N // 2
NC = 16
RC = M // NC
DEPTH = 4


def kernel(x):
    x2d = x.reshape(M, N)

    def body(
        x_ref,
        out_ref,
        recv_buf,
        sbuf,
        sstage,
        stage_sems,
        send_y,
        recv_y,
        copy_sem,
    ):
        my_x = lax.axis_index("x")
        my_y = lax.axis_index("y")
        peer_y = 1 - my_y

        send_col = peer_y * H
        keep_col = my_y * H

        barrier_sem = pltpu.get_barrier_semaphore()
        pl.semaphore_signal(
            barrier_sem,
            inc=1,
            device_id=(my_x, peer_y),
            device_id_type=pl.DeviceIdType.MESH,
        )
        pl.semaphore_wait(barrier_sem, 1)

        local = pltpu.make_async_copy(
            x_ref.at[:, pl.ds(keep_col, H)], out_ref, copy_sem
        )
        local.start()

        def stage_dma(c):
            return pltpu.make_async_copy(
                x_ref.at[pl.ds(c * RC, RC), pl.ds(send_col, H)],
                sstage.at[c % DEPTH],
                stage_sems.at[c % DEPTH],
            )

        dmas = [stage_dma(c) for c in range(NC)]
        for c in range(min(DEPTH, NC)):
            dmas[c].start()
        y_rdmas = []
        for c in range(NC):
            dmas[c].wait()
            if c >= DEPTH:
                y_rdmas[c - DEPTH].wait_send()
            sbuf[c % DEPTH] = sstage[c % DEPTH].astype(jnp.bfloat16)
            if c + DEPTH < NC:
                dmas[c + DEPTH].start()
            rd = pltpu.make_async_remote_copy(
                src_ref=sbuf.at[c % DEPTH],
                dst_ref=recv_buf.at[pl.ds(c * RC, RC), :],
                send_sem=send_y.at[c],
                recv_sem=recv_y.at[c],
                device_id=(my_x, peer_y),
                device_id_type=pl.DeviceIdType.MESH,
            )
            rd.start()
            y_rdmas.append(rd)

        local.wait()

        for c in range(NC):
            r0 = c * RC
            y_rdmas[c].wait_recv()
            out_ref[pl.ds(r0, RC), :] = (
                out_ref[pl.ds(r0, RC), :]
                + recv_buf[pl.ds(r0, RC), :].astype(jnp.float32)
            )

        for c in range(NC - DEPTH, NC):
            y_rdmas[c].wait_send()

    return pl.pallas_call(
        body,
        out_shape=jax.ShapeDtypeStruct((M, H), jnp.float32),
        in_specs=[pl.BlockSpec(memory_space=pltpu.MemorySpace.HBM)],
        out_specs=pl.BlockSpec(memory_space=pltpu.VMEM),
        scratch_shapes=[
            pltpu.VMEM((M, H), jnp.bfloat16),
            pltpu.VMEM((DEPTH, RC, H), jnp.bfloat16),
            pltpu.VMEM((DEPTH, RC, H), jnp.float32),
            pltpu.SemaphoreType.DMA((DEPTH,)),
            pltpu.SemaphoreType.DMA((NC,)),
            pltpu.SemaphoreType.DMA((NC,)),
            pltpu.SemaphoreType.DMA,
        ],
        compiler_params=pltpu.CompilerParams(
            collective_id=0, vmem_limit_bytes=62 * 1024 * 1024
        ),
    )(x2d)


# device time: 118090 ns/iter; 1.8640x vs baseline; 1.8640x over previous
import jax
import jax.numpy as jnp
from jax import lax
from jax.experimental import pallas as pl
from jax.experimental.pallas import tpu as pltpu

M = 8192
N = 2048
H = N // 2
MH = M // 2
NC = 16
RC = MH // NC
DEPTH = 4
LDEPTH = 6
RDEPTH = 4


def kernel(x):
    x2d = x.reshape(M, N)

    def body(
        x_ref,
        out_ref,
        recv_buf,
        sbuf,
        sstage,
        lstage,
        rbuf,
        stage_sems,
        local_sems,
        result_sems,
        send_y,
        recv_y,
        send_x,
        recv_x,
    ):
        my_x = lax.axis_index("x")
        my_y = lax.axis_index("y")
        peer_y = 1 - my_y
        peer_x = 1 - my_x

        send_col = peer_y * H
        keep_col = my_y * H
        my_row0 = my_x * MH
        ot_row0 = peer_x * MH

        barrier_sem = pltpu.get_barrier_semaphore()
        for dev in ((my_x, peer_y), (peer_x, my_y)):
            pl.semaphore_signal(
                barrier_sem,
                inc=1,
                device_id=dev,
                device_id_type=pl.DeviceIdType.MESH,
            )
        pl.semaphore_wait(barrier_sem, 2)

        def stage_dma(c):
            return pltpu.make_async_copy(
                x_ref.at[pl.ds(my_row0 + c * RC, RC), pl.ds(send_col, H)],
                sstage.at[c % DEPTH],
                stage_sems.at[c % DEPTH],
            )

        dmas = [stage_dma(c) for c in range(NC)]
        for c in range(min(DEPTH, NC)):
            dmas[c].start()
        y_rdmas = []
        for c in range(NC):
            dmas[c].wait()
            sbuf[pl.ds(c * RC, RC), :] = sstage[c % DEPTH].astype(jnp.bfloat16)
            if c + DEPTH < NC:
                dmas[c + DEPTH].start()
            rd = pltpu.make_async_remote_copy(
                src_ref=sbuf.at[pl.ds(c * RC, RC), :],
                dst_ref=recv_buf.at[pl.ds(my_row0 + c * RC, RC), :],
                send_sem=send_y.at[c],
                recv_sem=recv_y.at[c],
                device_id=(my_x, peer_y),
                device_id_type=pl.DeviceIdType.MESH,
            )
            rd.start()
            y_rdmas.append(rd)

        def local_row(k):
            if k < NC:
                return my_row0 + k * RC
            return ot_row0 + (k - NC) * RC

        def local_dma(k):
            return pltpu.make_async_copy(
                x_ref.at[pl.ds(local_row(k), RC), pl.ds(keep_col, H)],
                lstage.at[k % LDEPTH],
                local_sems.at[k % LDEPTH],
            )

        local_dmas = [local_dma(k) for k in range(2 * NC)]
        n_issued = [0]

        def issue_local():
            if n_issued[0] < 2 * NC:
                local_dmas[n_issued[0]].start()
                n_issued[0] += 1

        for _ in range(min(LDEPTH, 2 * NC)):
            issue_local()

        result_copies = []

        def process(k, r0):
            local_dmas[k].wait()
            res = (
                lstage[k % LDEPTH]
                + recv_buf[pl.ds(r0, RC), :].astype(jnp.float32)
            ).astype(jnp.bfloat16)
            slot = k % RDEPTH
            if k >= RDEPTH:
                result_copies[k - RDEPTH].wait()
            rbuf[slot] = res
            cp = pltpu.make_async_copy(
                rbuf.at[slot],
                out_ref.at[pl.ds(r0, RC), :],
                result_sems.at[slot],
            )
            cp.start()
            result_copies.append(cp)
            issue_local()

        x_rdmas = []
        for c in range(NC):
            r0 = my_row0 + c * RC
            y_rdmas[c].wait_recv()
            rd = pltpu.make_async_remote_copy(
                src_ref=recv_buf.at[pl.ds(r0, RC), :],
                dst_ref=recv_buf.at[pl.ds(r0, RC), :],
                send_sem=send_x.at[c],
                recv_sem=recv_x.at[c],
                device_id=(peer_x, my_y),
                device_id_type=pl.DeviceIdType.MESH,
            )
            rd.start()
            x_rdmas.append(rd)
            process(c, r0)

        for c in range(NC):
            x_rdmas[c].wait_recv()
            process(NC + c, ot_row0 + c * RC)

        for c in range(NC):
            y_rdmas[c].wait_send()
            x_rdmas[c].wait_send()
        for k in range(2 * NC - RDEPTH, 2 * NC):
            result_copies[k].wait()

    return pl.pallas_call(
        body,
        out_shape=jax.ShapeDtypeStruct((M, H), jnp.bfloat16),
        in_specs=[pl.BlockSpec(memory_space=pltpu.MemorySpace.HBM)],
        out_specs=pl.BlockSpec(memory_space=pltpu.MemorySpace.HBM),
        scratch_shapes=[
            pltpu.VMEM((M, H), jnp.bfloat16),
            pltpu.VMEM((MH, H), jnp.bfloat16),
            pltpu.VMEM((DEPTH, RC, H), jnp.float32),
            pltpu.VMEM((LDEPTH, RC, H), jnp.float32),
            pltpu.VMEM((RDEPTH, RC, H), jnp.bfloat16),
            pltpu.SemaphoreType.DMA((DEPTH,)),
            pltpu.SemaphoreType.DMA((LDEPTH,)),
            pltpu.SemaphoreType.DMA((RDEPTH,)),
            pltpu.SemaphoreType.DMA((NC,)),
            pltpu.SemaphoreType.DMA((NC,)),
            pltpu.SemaphoreType.DMA((NC,)),
            pltpu.SemaphoreType.DMA((NC,)),
        ],
        compiler_params=pltpu.CompilerParams(
            collective_id=0, vmem_limit_bytes=56 * 1024 * 1024
        ),
    )(x2d)
